# Initial kernel scaffold; baseline (speedup 1.0000x reference)
import jax
import jax.numpy as jnp
from jax.experimental import pallas as pl


def kernel(x_emb, params):
    raise NotImplementedError("write your pallas kernel here")


# speedup vs baseline: 1.6148x; 1.6148x over previous
import functools

import jax
import jax.numpy as jnp
from jax import lax
from jax.experimental import pallas as pl
from jax.experimental.pallas import tpu as pltpu
from jax.experimental.pallas import tpu_sc as plsc

D = 768
H = 12
DH = 64
E = 8
F = 3072


def _erf(x):
    return lax.erf(x)


def _gelu(x):
    return 0.5 * x * (1.0 + _erf(x * 0.7071067811865476))


def _ln_body(nin, x_ref, r_ref, g_ref, b_ref, o_ref):
    x = x_ref[...]
    if nin == 2:
        x = x + r_ref[...]
    m = jnp.mean(x, axis=-1, keepdims=True)
    v = jnp.mean((x - m) ** 2, axis=-1, keepdims=True)
    o_ref[...] = (x - m) * lax.rsqrt(v + 1e-5) * g_ref[...] + b_ref[...]


def _ln(x, g, b, residual=None):
    n = x.shape[0]
    bn = min(n, 256)
    nin = 2 if residual is not None else 1
    args = [x] + ([residual] if residual is not None else [])
    specs = [pl.BlockSpec((bn, D), lambda i: (i, 0))] * nin
    specs += [pl.BlockSpec((1, D), lambda i: (0, 0))] * 2
    body = functools.partial(_ln_body, nin)
    if nin == 1:
        body = lambda x_ref, g_ref, b_ref, o_ref: _ln_body(
            1, x_ref, None, g_ref, b_ref, o_ref)
    return pl.pallas_call(
        body,
        grid=(n // bn,),
        in_specs=specs,
        out_specs=pl.BlockSpec((bn, D), lambda i: (i, 0)),
        out_shape=jax.ShapeDtypeStruct((n, D), jnp.float32),
    )(*args, g.reshape(1, D), b.reshape(1, D))


def _mm_body(has_res, x_ref, w_ref, b_ref, *rest):
    if has_res:
        r_ref, o_ref = rest
    else:
        (o_ref,) = rest
    acc = lax.dot_general(x_ref[...], w_ref[...], (((1,), (1,)), ((), ())),
                          preferred_element_type=jnp.float32)
    acc = acc + b_ref[...]
    if has_res:
        acc = acc + rest[0][...]
    o_ref[...] = acc


def _matmul(x, w, b, bm, residual=None):
    n, k = x.shape
    m = w.shape[0]
    bn = min(n, 256)
    has_res = residual is not None
    in_specs = [
        pl.BlockSpec((bn, k), lambda j, i: (i, 0)),
        pl.BlockSpec((bm, k), lambda j, i: (j, 0)),
        pl.BlockSpec((1, bm), lambda j, i: (0, j)),
    ]
    args = [x, w, b.reshape(1, m)]
    if has_res:
        in_specs.append(pl.BlockSpec((bn, bm), lambda j, i: (i, j)))
        args.append(residual)
    return pl.pallas_call(
        functools.partial(_mm_body, has_res),
        grid=(m // bm, n // bn),
        in_specs=in_specs,
        out_specs=pl.BlockSpec((bn, bm), lambda j, i: (i, j)),
        out_shape=jax.ShapeDtypeStruct((n, m), jnp.float32),
    )(*args)


def _attn_body(bq, n, q_ref, k_ref, v_ref, o_ref):
    i = pl.program_id(1)
    q = q_ref[0]
    k = k_ref[0]
    v = v_ref[0]
    s = lax.dot_general(q, k, (((1,), (1,)), ((), ())),
                        preferred_element_type=jnp.float32) * (1.0 / 8.0)
    row = i * bq + lax.broadcasted_iota(jnp.int32, (bq, n), 0)
    col = lax.broadcasted_iota(jnp.int32, (bq, n), 1)
    s = jnp.where(col <= row, s, jnp.float32(-1e9))
    mx = jnp.max(s, axis=-1, keepdims=True)
    p = jnp.exp(s - mx)
    den = jnp.sum(p, axis=-1, keepdims=True)
    o = lax.dot_general(p, v, (((1,), (0,)), ((), ())),
                        preferred_element_type=jnp.float32)
    o_ref[0] = o / den


def _attn(q, k, v):
    n = q.shape[1]
    bq = min(n, 256)
    spec_full = pl.BlockSpec((1, n, DH), lambda h, i: (h, 0, 0))
    spec_q = pl.BlockSpec((1, bq, DH), lambda h, i: (h, i, 0))
    return pl.pallas_call(
        functools.partial(_attn_body, bq, n),
        grid=(H, n // bq),
        in_specs=[spec_q, spec_full, spec_full],
        out_specs=spec_q,
        out_shape=jax.ShapeDtypeStruct((H, n, DH), jnp.float32),
    )(q, k, v)


def _router_body(bn, nblocks, n, lg_ref, idx_ref, w_ref, st_ref):
    i = pl.program_id(0)
    lg = lg_ref[...]
    col = lax.broadcasted_iota(jnp.int32, (bn, 128), 1)
    lg = jnp.where(col < E, lg, jnp.float32(-1e30))
    mx = jnp.max(lg, axis=-1, keepdims=True)
    p = jnp.exp(lg - mx)
    probs = p / jnp.sum(p, axis=-1, keepdims=True)
    v0 = jnp.max(probs, axis=-1, keepdims=True)
    i0 = jnp.min(jnp.where(probs == v0, col, 9999), axis=-1, keepdims=True)
    oh0 = col == i0
    masked = jnp.where(oh0, jnp.float32(-1.0), probs)
    v1 = jnp.max(masked, axis=-1, keepdims=True)
    i1 = jnp.min(jnp.where(masked == v1, col, 9999), axis=-1, keepdims=True)
    oh1 = col == i1
    den = v0 + v1
    idx_ref[...] = jnp.concatenate([i0, i1], axis=1)
    w_ref[...] = jnp.concatenate([v0 / den, v1 / den], axis=1)

    psum = jnp.sum(probs, axis=0, keepdims=True)
    csum = jnp.sum(jnp.where(oh0 | oh1, 1.0, 0.0), axis=0, keepdims=True)

    @pl.when(i == 0)
    def _():
        st_ref[...] = jnp.zeros_like(st_ref)

    st_ref[0:1, :] += psum
    st_ref[1:2, :] += csum

    @pl.when(i == nblocks - 1)
    def _():
        ps = st_ref[0:1, :]
        cs = st_ref[1:2, :]
        bal = jnp.float32(E) * jnp.sum(ps * cs) / jnp.float32(n * n)
        st_ref[2:3, :] = jnp.full((1, 128), bal, jnp.float32)


def _router(logits, n):
    bn = min(n, 256)
    nb = n // bn
    return pl.pallas_call(
        functools.partial(_router_body, bn, nb, n),
        grid=(nb,),
        in_specs=[pl.BlockSpec((bn, 128), lambda i: (i, 0))],
        out_specs=[
            pl.BlockSpec((bn, 2), lambda i: (i, 0)),
            pl.BlockSpec((bn, 2), lambda i: (i, 0)),
            pl.BlockSpec((8, 128), lambda i: (0, 0)),
        ],
        out_shape=[
            jax.ShapeDtypeStruct((n, 2), jnp.int32),
            jax.ShapeDtypeStruct((n, 2), jnp.float32),
            jax.ShapeDtypeStruct((8, 128), jnp.float32),
        ],
    )(logits)


_GDN = lax.GatherDimensionNumbers(
    offset_dims=(), collapsed_slice_dims=(0,), start_index_map=(0,))


def _lane_gather(x, idx):
    return lax.gather(x, idx[:, None], _GDN, slice_sizes=(1,),
                      mode=lax.GatherScatterMode.PROMISE_IN_BOUNDS)


def _lane_cumsum(x, lanes):
    for k in (1, 2, 4, 8):
        y = _lane_gather(x, jnp.maximum(lanes - k, 0))
        x = x + jnp.where(lanes >= k, y, 0)
    return x
def _dispatch(idx_flat, ntot):
    mesh = plsc.VectorSubcoreMesh(core_axis_name="c", subcore_axis_name="s")
    nv = ntot // 16

    @functools.partial(
        pl.kernel,
        mesh=mesh,
        out_type=[
            jax.ShapeDtypeStruct((ntot,), jnp.int32),
            jax.ShapeDtypeStruct((ntot,), jnp.int32),
            jax.ShapeDtypeStruct((16,), jnp.int32),
        ],
        scratch_types=[
            pltpu.VMEM((ntot,), jnp.int32),
            pltpu.VMEM((ntot + 16,), jnp.int32),
            pltpu.VMEM((ntot,), jnp.int32),
            pltpu.VMEM((16,), jnp.int32),
        ],
        compiler_params=pltpu.CompilerParams(needs_layout_passes=False),
    )
    def disp(idx_hbm, st_hbm, pos_hbm, off_hbm, idx_v, st_v, pos_v, off_v):
        wid = lax.axis_index("s") * 2 + lax.axis_index("c")

        @pl.when(wid == 0)
        def _():
            pltpu.sync_copy(idx_hbm, idx_v)
            lanes = lax.iota(jnp.int32, 16)
            offvec = jnp.full((16,), ntot, jnp.int32)
            ptr = jnp.int32(0)
            for e in range(E):
                offvec = jnp.where(lanes == e, ptr, offvec)

                def body(c, ptr):
                    v = idx_v[pl.ds(c * 16, 16)]
                    m = v == e
                    asn = c * 16 + lanes
                    mi = jnp.where(m, jnp.int32(1), jnp.int32(0))
                    pref = _lane_cumsum(mi, lanes)
                    dest = jnp.where(m, ptr + pref - 1, ntot + lanes)
                    plsc.store_scatter(st_v, [dest], asn)
                    return ptr + jnp.max(pref)

                ptr = lax.fori_loop(0, nv, body, ptr)
            off_v[...] = offvec

            def inv(c, _):
                a = st_v[pl.ds(c * 16, 16)]
                plsc.store_scatter(pos_v, [a], c * 16 + lanes)
                return _

            lax.fori_loop(0, nv, inv, jnp.int32(0))

            def toks(c, _):
                st_v[pl.ds(c * 16, 16)] = jnp.right_shift(
                    st_v[pl.ds(c * 16, 16)], 1)
                return _

            lax.fori_loop(0, nv, toks, jnp.int32(0))
            pltpu.sync_copy(st_v.at[pl.ds(0, ntot)], st_hbm)
            pltpu.sync_copy(pos_v, pos_hbm)
            pltpu.sync_copy(off_v, off_hbm)

    return disp(idx_flat)


def _gather(table, idx):
    v, d = table.shape
    b = idx.shape[0]
    nw = 32
    bpw = b // nw
    mesh = plsc.VectorSubcoreMesh(core_axis_name="c", subcore_axis_name="s")

    @functools.partial(
        pl.kernel,
        mesh=mesh,
        out_type=jax.ShapeDtypeStruct((b, d), jnp.float32),
        scratch_types=[
            pltpu.VMEM((bpw,), jnp.int32),
            pltpu.VMEM((bpw, d), jnp.float32),
            pltpu.SemaphoreType.DMA,
        ],
        compiler_params=pltpu.CompilerParams(needs_layout_passes=False),
    )
    def gk(tab_hbm, idx_hbm, out_hbm, idx_v, rows_v, sem):
        wid = lax.axis_index("s") * 2 + lax.axis_index("c")
        base = wid * bpw
        pltpu.sync_copy(idx_hbm.at[pl.ds(base, bpw)], idx_v)
        pltpu.async_copy(tab_hbm.at[idx_v], rows_v, sem).wait()
        pltpu.sync_copy(rows_v, out_hbm.at[pl.ds(base, bpw)])

    return gk(table, idx)


def _gmm_body(bm, off_ref, x_ref, w1_ref, b1_ref, w2_ref, b2_ref,
              o_ref, acc_ref):
    g = pl.program_id(0)
    m = pl.program_id(1)
    row0 = m * bm
    og = off_ref[g]
    og1 = off_ref[g + 1]

    @pl.when(g == 0)
    def _():
        acc_ref[pl.ds(row0, bm), :] = jnp.zeros((bm, D), jnp.float32)

    @pl.when((og1 > row0) & (og < row0 + bm))
    def _():
        x = x_ref[...]
        h = lax.dot_general(x, w1_ref[0], (((1,), (1,)), ((), ())),
                            preferred_element_type=jnp.float32)
        h = _gelu(h + b1_ref[0])
        y = lax.dot_general(h, w2_ref[0], (((1,), (1,)), ((), ())),
                            preferred_element_type=jnp.float32)
        y = y + b2_ref[0]
        rows = row0 + lax.broadcasted_iota(jnp.int32, (bm, 1), 0)
        msk = jnp.where((rows >= og) & (rows < og1), 1.0, 0.0)
        acc_ref[pl.ds(row0, bm), :] += y * msk

    o_ref[...] = acc_ref[pl.ds(row0, bm), :]


def _gmm(xs, w1, b1, w2, b2, offsets, ntot):
    bm = min(ntot, 512)
    mt = ntot // bm
    grid_spec = pltpu.PrefetchScalarGridSpec(
        num_scalar_prefetch=1,
        grid=(E, mt),
        in_specs=[
            pl.BlockSpec((bm, D), lambda g, m, off: (m, 0)),
            pl.BlockSpec((1, F, D), lambda g, m, off: (g, 0, 0)),
            pl.BlockSpec((1, 1, F), lambda g, m, off: (g, 0, 0)),
            pl.BlockSpec((1, D, F), lambda g, m, off: (g, 0, 0)),
            pl.BlockSpec((1, 1, D), lambda g, m, off: (g, 0, 0)),
        ],
        out_specs=pl.BlockSpec((bm, D), lambda g, m, off: (m, 0)),
        scratch_shapes=[pltpu.VMEM((ntot, D), jnp.float32)],
    )
    return pl.pallas_call(
        functools.partial(_gmm_body, bm),
        grid_spec=grid_spec,
        out_shape=jax.ShapeDtypeStruct((ntot, D), jnp.float32),
        compiler_params=pltpu.CompilerParams(
            vmem_limit_bytes=100 * 1024 * 1024),
    )(offsets, xs, w1, b1.reshape(E, 1, F), w2, b2.reshape(E, 1, D))


def _combine_body(x_ref, y_ref, w_ref, o_ref):
    w0 = w_ref[:, 0:1]
    w1 = w_ref[:, 1:2]
    o_ref[...] = x_ref[...] + w0 * y_ref[:, 0, :] + w1 * y_ref[:, 1, :]


def _combine(x, y2, w):
    n = x.shape[0]
    bn = min(n, 256)
    return pl.pallas_call(
        _combine_body,
        grid=(n // bn,),
        in_specs=[
            pl.BlockSpec((bn, D), lambda i: (i, 0)),
            pl.BlockSpec((bn, 2, D), lambda i: (i, 0, 0)),
            pl.BlockSpec((bn, 2), lambda i: (i, 0)),
        ],
        out_specs=pl.BlockSpec((bn, D), lambda i: (i, 0)),
        out_shape=jax.ShapeDtypeStruct((n, D), jnp.float32),
    )(x, y2, w)


def _add_body(a_ref, b_ref, o_ref):
    o_ref[...] = a_ref[...] + b_ref[...]


def _addk(a, b):
    n = a.shape[0]
    bn = min(n, 256)
    return pl.pallas_call(
        _add_body,
        grid=(n // bn,),
        in_specs=[pl.BlockSpec((bn, D), lambda i: (i, 0))] * 2,
        out_specs=pl.BlockSpec((bn, D), lambda i: (i, 0)),
        out_shape=jax.ShapeDtypeStruct((n, D), jnp.float32),
    )(a, b)


def _layer(x, p, n):
    h1 = _ln(x, p['ln1_g'], p['ln1_b'])
    qkv = _matmul(h1, p['attn_in_w'], p['attn_in_b'], bm=768)
    q = qkv[:, :D].reshape(n, H, DH).transpose(1, 0, 2)
    k = qkv[:, D:2 * D].reshape(n, H, DH).transpose(1, 0, 2)
    v = qkv[:, 2 * D:].reshape(n, H, DH).transpose(1, 0, 2)
    o = _attn(q, k, v).transpose(1, 0, 2).reshape(n, D)
    x = _matmul(o, p['attn_out_w'], p['attn_out_b'], bm=768, residual=x)
    h2 = _ln(x, p['ln2_g'], p['ln2_b'])
    rw = jnp.zeros((128, D), jnp.float32).at[:E].set(p['router_w'])
    logits = _matmul(h2, rw, jnp.zeros((128,), jnp.float32), bm=128)
    idx2, w2, stats = _router(logits, n)
    ntot = 2 * n
    st, pos, off = _dispatch(idx2.reshape(ntot), ntot)
    xs = _gather(h2, st)
    ys = _gmm(xs, p['e_w1'], p['e_b1'], p['e_w2'], p['e_b2'], off, ntot)
    y2 = _gather(ys, pos).reshape(n, 2, D)
    x = _combine(x, y2, w2)
    return x, stats[2, 0]


def kernel(x_emb, params):
    b, t, _ = x_emb.shape
    x = _addk(x_emb[0], params['pos'][:t])
    bal = jnp.float32(0.0)
    h = x
    for p in params['local']:
        h, bl = _layer(h, p, t)
        bal = bal + bl
    syn = h[15::16]
    s = syn.shape[0]
    g = syn
    for p in params['global']:
        g, bl = _layer(g, p, s)
        bal = bal + bl
    rep = jnp.repeat(g, 16, axis=0)
    out = _ln(h, params['ln_g'], params['ln_b'], residual=rep)
    logits = _matmul(out, params['head_w'], jnp.zeros((8192,), jnp.float32),
                     bm=512)
    return logits[None], bal


# speedup vs baseline: 2.0051x; 1.2417x over previous
import functools

import jax
import jax.numpy as jnp
from jax import lax
from jax.experimental import pallas as pl
from jax.experimental.pallas import tpu as pltpu
from jax.experimental.pallas import tpu_sc as plsc

D = 768
H = 12
DH = 64
E = 8
F = 3072


def _erf(x):
    return lax.erf(x)


def _gelu(x):
    return 0.5 * x * (1.0 + _erf(x * 0.7071067811865476))


def _ln_body(nin, x_ref, r_ref, g_ref, b_ref, o_ref):
    x = x_ref[...]
    if nin == 2:
        x = x + r_ref[...]
    m = jnp.mean(x, axis=-1, keepdims=True)
    v = jnp.mean((x - m) ** 2, axis=-1, keepdims=True)
    o_ref[...] = (x - m) * lax.rsqrt(v + 1e-5) * g_ref[...] + b_ref[...]


def _ln(x, g, b, residual=None):
    n = x.shape[0]
    bn = min(n, 256)
    nin = 2 if residual is not None else 1
    args = [x] + ([residual] if residual is not None else [])
    specs = [pl.BlockSpec((bn, D), lambda i: (i, 0))] * nin
    specs += [pl.BlockSpec((1, D), lambda i: (0, 0))] * 2
    body = functools.partial(_ln_body, nin)
    if nin == 1:
        body = lambda x_ref, g_ref, b_ref, o_ref: _ln_body(
            1, x_ref, None, g_ref, b_ref, o_ref)
    return pl.pallas_call(
        body,
        grid=(n // bn,),
        in_specs=specs,
        out_specs=pl.BlockSpec((bn, D), lambda i: (i, 0)),
        out_shape=jax.ShapeDtypeStruct((n, D), jnp.float32),
    )(*args, g.reshape(1, D), b.reshape(1, D))


def _mm_body(has_res, x_ref, w_ref, b_ref, *rest):
    if has_res:
        r_ref, o_ref = rest
    else:
        (o_ref,) = rest
    acc = lax.dot_general(x_ref[...], w_ref[...], (((1,), (1,)), ((), ())),
                          preferred_element_type=jnp.float32)
    acc = acc + b_ref[...]
    if has_res:
        acc = acc + rest[0][...]
    o_ref[...] = acc


def _matmul(x, w, b, bm, residual=None):
    n, k = x.shape
    m = w.shape[0]
    bn = min(n, 256)
    has_res = residual is not None
    in_specs = [
        pl.BlockSpec((bn, k), lambda j, i: (i, 0)),
        pl.BlockSpec((bm, k), lambda j, i: (j, 0)),
        pl.BlockSpec((1, bm), lambda j, i: (0, j)),
    ]
    args = [x, w, b.reshape(1, m)]
    if has_res:
        in_specs.append(pl.BlockSpec((bn, bm), lambda j, i: (i, j)))
        args.append(residual)
    return pl.pallas_call(
        functools.partial(_mm_body, has_res),
        grid=(m // bm, n // bn),
        in_specs=in_specs,
        out_specs=pl.BlockSpec((bn, bm), lambda j, i: (i, j)),
        out_shape=jax.ShapeDtypeStruct((n, m), jnp.float32),
    )(*args)


def _attn_body(bq, n, q_ref, k_ref, v_ref, o_ref):
    i = pl.program_id(1)
    row = i * bq + lax.broadcasted_iota(jnp.int32, (bq, n), 0)
    col = lax.broadcasted_iota(jnp.int32, (bq, n), 1)
    causal = col <= row
    for h in range(2):
        q = q_ref[h]
        k = k_ref[h]
        v = v_ref[h]
        s = lax.dot_general(q, k, (((1,), (1,)), ((), ())),
                            preferred_element_type=jnp.float32) * (1.0 / 8.0)
        s = jnp.where(causal, s, jnp.float32(-1e9))
        mx = jnp.max(s, axis=-1, keepdims=True)
        p = jnp.exp(s - mx)
        den = jnp.sum(p, axis=-1, keepdims=True)
        o = lax.dot_general(p, v, (((1,), (0,)), ((), ())),
                            preferred_element_type=jnp.float32)
        o_ref[h] = o / den


def _attn(q, k, v):
    n = q.shape[1]
    bq = min(n, 512)
    spec_full = pl.BlockSpec((2, n, DH), lambda h, i: (h, 0, 0))
    spec_q = pl.BlockSpec((2, bq, DH), lambda h, i: (h, i, 0))
    return pl.pallas_call(
        functools.partial(_attn_body, bq, n),
        grid=(H // 2, n // bq),
        in_specs=[spec_q, spec_full, spec_full],
        out_specs=spec_q,
        out_shape=jax.ShapeDtypeStruct((H, n, DH), jnp.float32),
    )(q, k, v)


def _router_body(bn, nblocks, n, lg_ref, idx_ref, w_ref, st_ref):
    i = pl.program_id(0)
    lg = lg_ref[...]
    col = lax.broadcasted_iota(jnp.int32, (bn, 128), 1)
    lg = jnp.where(col < E, lg, jnp.float32(-1e30))
    mx = jnp.max(lg, axis=-1, keepdims=True)
    p = jnp.exp(lg - mx)
    probs = p / jnp.sum(p, axis=-1, keepdims=True)
    v0 = jnp.max(probs, axis=-1, keepdims=True)
    i0 = jnp.min(jnp.where(probs == v0, col, 9999), axis=-1, keepdims=True)
    oh0 = col == i0
    masked = jnp.where(oh0, jnp.float32(-1.0), probs)
    v1 = jnp.max(masked, axis=-1, keepdims=True)
    i1 = jnp.min(jnp.where(masked == v1, col, 9999), axis=-1, keepdims=True)
    oh1 = col == i1
    den = v0 + v1
    idx_ref[...] = jnp.concatenate([i0, i1], axis=1)
    w_ref[...] = jnp.concatenate([v0 / den, v1 / den], axis=1)

    psum = jnp.sum(probs, axis=0, keepdims=True)
    csum = jnp.sum(jnp.where(oh0 | oh1, 1.0, 0.0), axis=0, keepdims=True)

    @pl.when(i == 0)
    def _():
        st_ref[...] = jnp.zeros_like(st_ref)

    st_ref[0:1, :] += psum
    st_ref[1:2, :] += csum

    @pl.when(i == nblocks - 1)
    def _():
        ps = st_ref[0:1, :]
        cs = st_ref[1:2, :]
        bal = jnp.float32(E) * jnp.sum(ps * cs) / jnp.float32(n * n)
        st_ref[2:3, :] = jnp.full((1, 128), bal, jnp.float32)


def _router(logits, n):
    bn = min(n, 256)
    nb = n // bn
    return pl.pallas_call(
        functools.partial(_router_body, bn, nb, n),
        grid=(nb,),
        in_specs=[pl.BlockSpec((bn, 128), lambda i: (i, 0))],
        out_specs=[
            pl.BlockSpec((bn, 2), lambda i: (i, 0)),
            pl.BlockSpec((bn, 2), lambda i: (i, 0)),
            pl.BlockSpec((8, 128), lambda i: (0, 0)),
        ],
        out_shape=[
            jax.ShapeDtypeStruct((n, 2), jnp.int32),
            jax.ShapeDtypeStruct((n, 2), jnp.float32),
            jax.ShapeDtypeStruct((8, 128), jnp.float32),
        ],
    )(logits)


_GDN = lax.GatherDimensionNumbers(
    offset_dims=(), collapsed_slice_dims=(0,), start_index_map=(0,))


def _lane_gather(x, idx):
    return lax.gather(x, idx[:, None], _GDN, slice_sizes=(1,),
                      mode=lax.GatherScatterMode.PROMISE_IN_BOUNDS)


def _lane_cumsum(x, lanes):
    for k in (1, 2, 4, 8):
        y = _lane_gather(x, jnp.maximum(lanes - k, 0))
        x = x + jnp.where(lanes >= k, y, 0)
    return x
def _dispatch(idx_flat, ntot):
    mesh = plsc.VectorSubcoreMesh(core_axis_name="c", subcore_axis_name="s")
    nv = ntot // 16

    @functools.partial(
        pl.kernel,
        mesh=mesh,
        out_type=[
            jax.ShapeDtypeStruct((ntot,), jnp.int32),
            jax.ShapeDtypeStruct((ntot,), jnp.int32),
            jax.ShapeDtypeStruct((16,), jnp.int32),
        ],
        scratch_types=[
            pltpu.VMEM((ntot,), jnp.int32),
            pltpu.VMEM((ntot + 16,), jnp.int32),
            pltpu.VMEM((ntot,), jnp.int32),
            pltpu.VMEM((16,), jnp.int32),
        ],
        compiler_params=pltpu.CompilerParams(needs_layout_passes=False),
    )
    def disp(idx_hbm, st_hbm, pos_hbm, off_hbm, idx_v, st_v, pos_v, off_v):
        wid = lax.axis_index("s") * 2 + lax.axis_index("c")

        @pl.when(wid == 0)
        def _():
            pltpu.sync_copy(idx_hbm, idx_v)
            lanes = lax.iota(jnp.int32, 16)
            offvec = jnp.full((16,), ntot, jnp.int32)
            ptr = jnp.int32(0)
            for e in range(E):
                offvec = jnp.where(lanes == e, ptr, offvec)

                def body(c, ptr):
                    v = idx_v[pl.ds(c * 16, 16)]
                    m = v == e
                    asn = c * 16 + lanes
                    mi = jnp.where(m, jnp.int32(1), jnp.int32(0))
                    pref = _lane_cumsum(mi, lanes)
                    dest = jnp.where(m, ptr + pref - 1, ntot + lanes)
                    plsc.store_scatter(st_v, [dest], asn)
                    return ptr + jnp.max(pref)

                ptr = lax.fori_loop(0, nv, body, ptr)
            off_v[...] = offvec

            def inv(c, _):
                a = st_v[pl.ds(c * 16, 16)]
                plsc.store_scatter(pos_v, [a], c * 16 + lanes)
                return _

            lax.fori_loop(0, nv, inv, jnp.int32(0))

            def toks(c, _):
                st_v[pl.ds(c * 16, 16)] = jnp.right_shift(
                    st_v[pl.ds(c * 16, 16)], 1)
                return _

            lax.fori_loop(0, nv, toks, jnp.int32(0))
            pltpu.sync_copy(st_v.at[pl.ds(0, ntot)], st_hbm)
            pltpu.sync_copy(pos_v, pos_hbm)
            pltpu.sync_copy(off_v, off_hbm)

    return disp(idx_flat)


def _gather(table, idx):
    v, d = table.shape
    b = idx.shape[0]
    nw = 32
    bpw = b // nw
    mesh = plsc.VectorSubcoreMesh(core_axis_name="c", subcore_axis_name="s")

    @functools.partial(
        pl.kernel,
        mesh=mesh,
        out_type=jax.ShapeDtypeStruct((b, d), jnp.float32),
        scratch_types=[
            pltpu.VMEM((bpw,), jnp.int32),
            pltpu.VMEM((bpw, d), jnp.float32),
            pltpu.SemaphoreType.DMA,
        ],
        compiler_params=pltpu.CompilerParams(needs_layout_passes=False),
    )
    def gk(tab_hbm, idx_hbm, out_hbm, idx_v, rows_v, sem):
        wid = lax.axis_index("s") * 2 + lax.axis_index("c")
        base = wid * bpw
        pltpu.sync_copy(idx_hbm.at[pl.ds(base, bpw)], idx_v)
        pltpu.async_copy(tab_hbm.at[idx_v], rows_v, sem).wait()
        pltpu.sync_copy(rows_v, out_hbm.at[pl.ds(base, bpw)])

    return gk(table, idx)


def _gmm_body(bm, off_ref, mi_ref, gi_ref, x_ref, w1_ref, b1_ref,
              w2_ref, b2_ref, o_ref):
    t = pl.program_id(0)
    mi = mi_ref[t]
    g = gi_ref[t]
    row0 = mi * bm
    og = off_ref[g]
    og1 = off_ref[g + 1]
    mi_prev = mi_ref[jnp.maximum(t - 1, 0)]
    is_first = (t == 0) | (mi != mi_prev)

    @pl.when(is_first)
    def _():
        o_ref[...] = jnp.zeros((bm, D), jnp.float32)

    @pl.when((og1 > row0) & (og < row0 + bm))
    def _():
        x = x_ref[...]
        h = lax.dot_general(x, w1_ref[0], (((1,), (1,)), ((), ())),
                            preferred_element_type=jnp.float32)
        h = _gelu(h + b1_ref[0])
        y = lax.dot_general(h, w2_ref[0], (((1,), (1,)), ((), ())),
                            preferred_element_type=jnp.float32)
        y = y + b2_ref[0]
        rows = row0 + lax.broadcasted_iota(jnp.int32, (bm, 1), 0)
        msk = jnp.where((rows >= og) & (rows < og1), 1.0, 0.0)
        o_ref[...] += y * msk


def _gmm(xs, w1, b1, w2, b2, offsets, ntot):
    bm = min(ntot, 512)
    mt = ntot // bm
    nt = mt + E
    m_grid = jnp.broadcast_to(jnp.arange(mt, dtype=jnp.int32)[:, None],
                              (mt, E)).reshape(-1)
    g_grid = jnp.broadcast_to(jnp.arange(E, dtype=jnp.int32)[None, :],
                              (mt, E)).reshape(-1)
    lo = offsets[:E][None, :]
    hi = offsets[1:E + 1][None, :]
    mrow = jnp.arange(mt, dtype=jnp.int32)[:, None]
    valid = ((hi > mrow * bm) & (lo < (mrow + 1) * bm)).reshape(-1)
    r = jnp.cumsum(valid.astype(jnp.int32)) - 1
    slots = jnp.where(valid, r, nt)
    m_ids = jnp.full((nt + 1,), mt - 1, jnp.int32).at[slots].set(
        m_grid, mode='drop')[:nt]
    g_ids = jnp.full((nt + 1,), E, jnp.int32).at[slots].set(
        g_grid, mode='drop')[:nt]
    grid_spec = pltpu.PrefetchScalarGridSpec(
        num_scalar_prefetch=3,
        grid=(nt,),
        in_specs=[
            pl.BlockSpec((bm, D), lambda t, off, mi, gi: (mi[t], 0)),
            pl.BlockSpec((1, F, D),
                         lambda t, off, mi, gi: (jnp.minimum(gi[t], E - 1),
                                                 0, 0)),
            pl.BlockSpec((1, 1, F),
                         lambda t, off, mi, gi: (jnp.minimum(gi[t], E - 1),
                                                 0, 0)),
            pl.BlockSpec((1, D, F),
                         lambda t, off, mi, gi: (jnp.minimum(gi[t], E - 1),
                                                 0, 0)),
            pl.BlockSpec((1, 1, D),
                         lambda t, off, mi, gi: (jnp.minimum(gi[t], E - 1),
                                                 0, 0)),
        ],
        out_specs=pl.BlockSpec((bm, D), lambda t, off, mi, gi: (mi[t], 0)),
    )
    return pl.pallas_call(
        functools.partial(_gmm_body, bm),
        grid_spec=grid_spec,
        out_shape=jax.ShapeDtypeStruct((ntot, D), jnp.float32),
        compiler_params=pltpu.CompilerParams(
            vmem_limit_bytes=100 * 1024 * 1024),
    )(offsets, m_ids, g_ids, xs, w1, b1.reshape(E, 1, F), w2,
      b2.reshape(E, 1, D))


def _combine_body(x_ref, y_ref, w_ref, o_ref):
    w0 = w_ref[:, 0:1]
    w1 = w_ref[:, 1:2]
    o_ref[...] = x_ref[...] + w0 * y_ref[:, 0, :] + w1 * y_ref[:, 1, :]


def _combine(x, y2, w):
    n = x.shape[0]
    bn = min(n, 256)
    return pl.pallas_call(
        _combine_body,
        grid=(n // bn,),
        in_specs=[
            pl.BlockSpec((bn, D), lambda i: (i, 0)),
            pl.BlockSpec((bn, 2, D), lambda i: (i, 0, 0)),
            pl.BlockSpec((bn, 2), lambda i: (i, 0)),
        ],
        out_specs=pl.BlockSpec((bn, D), lambda i: (i, 0)),
        out_shape=jax.ShapeDtypeStruct((n, D), jnp.float32),
    )(x, y2, w)


def _add_body(a_ref, b_ref, o_ref):
    o_ref[...] = a_ref[...] + b_ref[...]


def _addk(a, b):
    n = a.shape[0]
    bn = min(n, 256)
    return pl.pallas_call(
        _add_body,
        grid=(n // bn,),
        in_specs=[pl.BlockSpec((bn, D), lambda i: (i, 0))] * 2,
        out_specs=pl.BlockSpec((bn, D), lambda i: (i, 0)),
        out_shape=jax.ShapeDtypeStruct((n, D), jnp.float32),
    )(a, b)


def _layer(x, p, n):
    h1 = _ln(x, p['ln1_g'], p['ln1_b'])
    qkv = _matmul(h1, p['attn_in_w'], p['attn_in_b'], bm=768)
    q = qkv[:, :D].reshape(n, H, DH).transpose(1, 0, 2)
    k = qkv[:, D:2 * D].reshape(n, H, DH).transpose(1, 0, 2)
    v = qkv[:, 2 * D:].reshape(n, H, DH).transpose(1, 0, 2)
    o = _attn(q, k, v).transpose(1, 0, 2).reshape(n, D)
    x = _matmul(o, p['attn_out_w'], p['attn_out_b'], bm=768, residual=x)
    h2 = _ln(x, p['ln2_g'], p['ln2_b'])
    rw = jnp.zeros((128, D), jnp.float32).at[:E].set(p['router_w'])
    logits = _matmul(h2, rw, jnp.zeros((128,), jnp.float32), bm=128)
    idx2, w2, stats = _router(logits, n)
    ntot = 2 * n
    st, pos, off = _dispatch(idx2.reshape(ntot), ntot)
    xs = _gather(h2, st)
    ys = _gmm(xs, p['e_w1'], p['e_b1'], p['e_w2'], p['e_b2'], off, ntot)
    y2 = _gather(ys, pos).reshape(n, 2, D)
    x = _combine(x, y2, w2)
    return x, stats[2, 0]


def kernel(x_emb, params):
    b, t, _ = x_emb.shape
    x = _addk(x_emb[0], params['pos'][:t])
    bal = jnp.float32(0.0)
    h = x
    for p in params['local']:
        h, bl = _layer(h, p, t)
        bal = bal + bl
    syn = h[15::16]
    s = syn.shape[0]
    g = syn
    for p in params['global']:
        g, bl = _layer(g, p, s)
        bal = bal + bl
    rep = jnp.repeat(g, 16, axis=0)
    out = _ln(h, params['ln_g'], params['ln_b'], residual=rep)
    logits = _matmul(out, params['head_w'], jnp.zeros((8192,), jnp.float32),
                     bm=512)
    return logits[None], bal


# speedup vs baseline: 2.1179x; 1.0563x over previous
import functools

import jax
import jax.numpy as jnp
from jax import lax
from jax.experimental import pallas as pl
from jax.experimental.pallas import tpu as pltpu
from jax.experimental.pallas import tpu_sc as plsc

D = 768
H = 12
DH = 64
E = 8
F = 3072


def _erf(x):
    return lax.erf(x)


def _gelu(x):
    return 0.5 * x * (1.0 + _erf(x * 0.7071067811865476))


def _ln_body(nin, x_ref, r_ref, g_ref, b_ref, o_ref):
    x = x_ref[...]
    if nin == 2:
        x = x + r_ref[...]
    m = jnp.mean(x, axis=-1, keepdims=True)
    v = jnp.mean((x - m) ** 2, axis=-1, keepdims=True)
    o_ref[...] = (x - m) * lax.rsqrt(v + 1e-5) * g_ref[...] + b_ref[...]


def _ln(x, g, b, residual=None):
    n = x.shape[0]
    bn = min(n, 256)
    nin = 2 if residual is not None else 1
    args = [x] + ([residual] if residual is not None else [])
    specs = [pl.BlockSpec((bn, D), lambda i: (i, 0))] * nin
    specs += [pl.BlockSpec((1, D), lambda i: (0, 0))] * 2
    body = functools.partial(_ln_body, nin)
    if nin == 1:
        body = lambda x_ref, g_ref, b_ref, o_ref: _ln_body(
            1, x_ref, None, g_ref, b_ref, o_ref)
    return pl.pallas_call(
        body,
        grid=(n // bn,),
        in_specs=specs,
        out_specs=pl.BlockSpec((bn, D), lambda i: (i, 0)),
        out_shape=jax.ShapeDtypeStruct((n, D), jnp.float32),
    )(*args, g.reshape(1, D), b.reshape(1, D))


def _mm_body(has_res, x_ref, w_ref, b_ref, *rest):
    if has_res:
        r_ref, o_ref = rest
    else:
        (o_ref,) = rest
    acc = lax.dot_general(x_ref[...], w_ref[...], (((1,), (1,)), ((), ())),
                          preferred_element_type=jnp.float32)
    acc = acc + b_ref[...]
    if has_res:
        acc = acc + rest[0][...]
    o_ref[...] = acc


def _matmul(x, w, b, bm, residual=None):
    n, k = x.shape
    m = w.shape[0]
    bn = min(n, 512)
    has_res = residual is not None
    in_specs = [
        pl.BlockSpec((bn, k), lambda j, i: (i, 0)),
        pl.BlockSpec((bm, k), lambda j, i: (j, 0)),
        pl.BlockSpec((1, bm), lambda j, i: (0, j)),
    ]
    args = [x, w, b.reshape(1, m)]
    if has_res:
        in_specs.append(pl.BlockSpec((bn, bm), lambda j, i: (i, j)))
        args.append(residual)
    return pl.pallas_call(
        functools.partial(_mm_body, has_res),
        grid=(m // bm, n // bn),
        in_specs=in_specs,
        out_specs=pl.BlockSpec((bn, bm), lambda j, i: (i, j)),
        out_shape=jax.ShapeDtypeStruct((n, m), jnp.float32),
    )(*args)


def _attn_body(bq, n, q_ref, k_ref, v_ref, o_ref):
    i = pl.program_id(1)
    row = i * bq + lax.broadcasted_iota(jnp.int32, (bq, n), 0)
    col = lax.broadcasted_iota(jnp.int32, (bq, n), 1)
    causal = col <= row
    for h in range(2):
        q = q_ref[h]
        k = k_ref[h]
        v = v_ref[h]
        s = lax.dot_general(q, k, (((1,), (1,)), ((), ())),
                            preferred_element_type=jnp.float32) * (1.0 / 8.0)
        s = jnp.where(causal, s, jnp.float32(-1e9))
        mx = jnp.max(s, axis=-1, keepdims=True)
        p = jnp.exp(s - mx)
        den = jnp.sum(p, axis=-1, keepdims=True)
        o = lax.dot_general(p, v, (((1,), (0,)), ((), ())),
                            preferred_element_type=jnp.float32)
        o_ref[h] = o / den


def _attn(q, k, v):
    n = q.shape[1]
    bq = min(n, 512)
    spec_full = pl.BlockSpec((2, n, DH), lambda h, i: (h, 0, 0))
    spec_q = pl.BlockSpec((2, bq, DH), lambda h, i: (h, i, 0))
    return pl.pallas_call(
        functools.partial(_attn_body, bq, n),
        grid=(H // 2, n // bq),
        in_specs=[spec_q, spec_full, spec_full],
        out_specs=spec_q,
        out_shape=jax.ShapeDtypeStruct((H, n, DH), jnp.float32),
    )(q, k, v)


def _router_body(bn, nblocks, n, lg_ref, idx_ref, w_ref, st_ref):
    i = pl.program_id(0)
    lg = lg_ref[...]
    col = lax.broadcasted_iota(jnp.int32, (bn, 128), 1)
    lg = jnp.where(col < E, lg, jnp.float32(-1e30))
    mx = jnp.max(lg, axis=-1, keepdims=True)
    p = jnp.exp(lg - mx)
    probs = p / jnp.sum(p, axis=-1, keepdims=True)
    v0 = jnp.max(probs, axis=-1, keepdims=True)
    i0 = jnp.min(jnp.where(probs == v0, col, 9999), axis=-1, keepdims=True)
    oh0 = col == i0
    masked = jnp.where(oh0, jnp.float32(-1.0), probs)
    v1 = jnp.max(masked, axis=-1, keepdims=True)
    i1 = jnp.min(jnp.where(masked == v1, col, 9999), axis=-1, keepdims=True)
    oh1 = col == i1
    den = v0 + v1
    idx_ref[...] = jnp.concatenate([i0, i1], axis=1)
    w_ref[...] = jnp.concatenate([v0 / den, v1 / den], axis=1)

    psum = jnp.sum(probs, axis=0, keepdims=True)
    csum = jnp.sum(jnp.where(oh0 | oh1, 1.0, 0.0), axis=0, keepdims=True)

    @pl.when(i == 0)
    def _():
        st_ref[...] = jnp.zeros_like(st_ref)

    st_ref[0:1, :] += psum
    st_ref[1:2, :] += csum

    @pl.when(i == nblocks - 1)
    def _():
        ps = st_ref[0:1, :]
        cs = st_ref[1:2, :]
        bal = jnp.float32(E) * jnp.sum(ps * cs) / jnp.float32(n * n)
        st_ref[2:3, :] = jnp.full((1, 128), bal, jnp.float32)


def _router(logits, n):
    bn = min(n, 256)
    nb = n // bn
    return pl.pallas_call(
        functools.partial(_router_body, bn, nb, n),
        grid=(nb,),
        in_specs=[pl.BlockSpec((bn, 128), lambda i: (i, 0))],
        out_specs=[
            pl.BlockSpec((bn, 2), lambda i: (i, 0)),
            pl.BlockSpec((bn, 2), lambda i: (i, 0)),
            pl.BlockSpec((8, 128), lambda i: (0, 0)),
        ],
        out_shape=[
            jax.ShapeDtypeStruct((n, 2), jnp.int32),
            jax.ShapeDtypeStruct((n, 2), jnp.float32),
            jax.ShapeDtypeStruct((8, 128), jnp.float32),
        ],
    )(logits)


_GDN = lax.GatherDimensionNumbers(
    offset_dims=(), collapsed_slice_dims=(0,), start_index_map=(0,))


def _lane_gather(x, idx):
    return lax.gather(x, idx[:, None], _GDN, slice_sizes=(1,),
                      mode=lax.GatherScatterMode.PROMISE_IN_BOUNDS)


def _lane_cumsum(x, lanes):
    for k in (1, 2, 4, 8):
        y = _lane_gather(x, jnp.maximum(lanes - k, 0))
        x = x + jnp.where(lanes >= k, y, 0)
    return x
def _dispatch(idx_flat, ntot):
    mesh = plsc.VectorSubcoreMesh(core_axis_name="c", subcore_axis_name="s")
    nv = ntot // 16

    @functools.partial(
        pl.kernel,
        mesh=mesh,
        out_type=[
            jax.ShapeDtypeStruct((ntot,), jnp.int32),
            jax.ShapeDtypeStruct((ntot,), jnp.int32),
            jax.ShapeDtypeStruct((16,), jnp.int32),
        ],
        scratch_types=[
            pltpu.VMEM((ntot,), jnp.int32),
            pltpu.VMEM((ntot + 16,), jnp.int32),
            pltpu.VMEM((ntot,), jnp.int32),
            pltpu.VMEM((16,), jnp.int32),
        ],
        compiler_params=pltpu.CompilerParams(needs_layout_passes=False),
    )
    def disp(idx_hbm, st_hbm, pos_hbm, off_hbm, idx_v, st_v, pos_v, off_v):
        wid = lax.axis_index("s") * 2 + lax.axis_index("c")

        @pl.when(wid == 0)
        def _():
            pltpu.sync_copy(idx_hbm, idx_v)
            lanes = lax.iota(jnp.int32, 16)
            offvec = jnp.full((16,), ntot, jnp.int32)
            ptr = jnp.int32(0)
            for e in range(E):
                offvec = jnp.where(lanes == e, ptr, offvec)

                def body(c, ptr):
                    v = idx_v[pl.ds(c * 16, 16)]
                    m = v == e
                    asn = c * 16 + lanes
                    mi = jnp.where(m, jnp.int32(1), jnp.int32(0))
                    pref = _lane_cumsum(mi, lanes)
                    dest = jnp.where(m, ptr + pref - 1, ntot + lanes)
                    plsc.store_scatter(st_v, [dest], asn)
                    return ptr + jnp.max(pref)

                ptr = lax.fori_loop(0, nv, body, ptr)
            off_v[...] = offvec

            def inv(c, _):
                a = st_v[pl.ds(c * 16, 16)]
                plsc.store_scatter(pos_v, [a], c * 16 + lanes)
                st_v[pl.ds(c * 16, 16)] = jnp.right_shift(a, 1)
                return _

            lax.fori_loop(0, nv, inv, jnp.int32(0))
            pltpu.sync_copy(st_v.at[pl.ds(0, ntot)], st_hbm)
            pltpu.sync_copy(pos_v, pos_hbm)
            pltpu.sync_copy(off_v, off_hbm)

    return disp(idx_flat)


def _gather(table, idx):
    v, d = table.shape
    b = idx.shape[0]
    nw = 32
    bpw = b // nw
    mesh = plsc.VectorSubcoreMesh(core_axis_name="c", subcore_axis_name="s")

    @functools.partial(
        pl.kernel,
        mesh=mesh,
        out_type=jax.ShapeDtypeStruct((b, d), jnp.float32),
        scratch_types=[
            pltpu.VMEM((bpw,), jnp.int32),
            pltpu.VMEM((bpw, d), jnp.float32),
            pltpu.SemaphoreType.DMA,
        ],
        compiler_params=pltpu.CompilerParams(needs_layout_passes=False),
    )
    def gk(tab_hbm, idx_hbm, out_hbm, idx_v, rows_v, sem):
        wid = lax.axis_index("s") * 2 + lax.axis_index("c")
        base = wid * bpw
        pltpu.sync_copy(idx_hbm.at[pl.ds(base, bpw)], idx_v)
        pltpu.async_copy(tab_hbm.at[idx_v], rows_v, sem).wait()
        pltpu.sync_copy(rows_v, out_hbm.at[pl.ds(base, bpw)])

    return gk(table, idx)


def _gmm_body(bm, off_ref, mi_ref, gi_ref, x_ref, w1_ref, b1_ref,
              w2_ref, b2_ref, o_ref):
    t = pl.program_id(0)
    mi = mi_ref[t]
    g = gi_ref[t]
    row0 = mi * bm
    og = off_ref[g]
    og1 = off_ref[g + 1]
    mi_prev = mi_ref[jnp.maximum(t - 1, 0)]
    is_first = (t == 0) | (mi != mi_prev)

    @pl.when(is_first)
    def _():
        o_ref[...] = jnp.zeros((bm, D), jnp.float32)

    @pl.when((og1 > row0) & (og < row0 + bm))
    def _():
        x = x_ref[...]
        h = lax.dot_general(x, w1_ref[0], (((1,), (1,)), ((), ())),
                            preferred_element_type=jnp.float32)
        h = _gelu(h + b1_ref[0])
        y = lax.dot_general(h, w2_ref[0], (((1,), (1,)), ((), ())),
                            preferred_element_type=jnp.float32)
        y = y + b2_ref[0]
        rows = row0 + lax.broadcasted_iota(jnp.int32, (bm, 1), 0)
        msk = jnp.where((rows >= og) & (rows < og1), 1.0, 0.0)
        o_ref[...] += y * msk


def _gmm(xs, w1, b1, w2, b2, offsets, ntot):
    bm = min(ntot, 512)
    mt = ntot // bm
    nt = mt + E
    m_grid = jnp.broadcast_to(jnp.arange(mt, dtype=jnp.int32)[:, None],
                              (mt, E)).reshape(-1)
    g_grid = jnp.broadcast_to(jnp.arange(E, dtype=jnp.int32)[None, :],
                              (mt, E)).reshape(-1)
    lo = offsets[:E][None, :]
    hi = offsets[1:E + 1][None, :]
    mrow = jnp.arange(mt, dtype=jnp.int32)[:, None]
    valid = ((hi > mrow * bm) & (lo < (mrow + 1) * bm)).reshape(-1)
    r = jnp.cumsum(valid.astype(jnp.int32)) - 1
    slots = jnp.where(valid, r, nt)
    m_ids = jnp.full((nt + 1,), mt - 1, jnp.int32).at[slots].set(
        m_grid, mode='drop')[:nt]
    g_ids = jnp.full((nt + 1,), E, jnp.int32).at[slots].set(
        g_grid, mode='drop')[:nt]
    grid_spec = pltpu.PrefetchScalarGridSpec(
        num_scalar_prefetch=3,
        grid=(nt,),
        in_specs=[
            pl.BlockSpec((bm, D), lambda t, off, mi, gi: (mi[t], 0)),
            pl.BlockSpec((1, F, D),
                         lambda t, off, mi, gi: (jnp.minimum(gi[t], E - 1),
                                                 0, 0)),
            pl.BlockSpec((1, 1, F),
                         lambda t, off, mi, gi: (jnp.minimum(gi[t], E - 1),
                                                 0, 0)),
            pl.BlockSpec((1, D, F),
                         lambda t, off, mi, gi: (jnp.minimum(gi[t], E - 1),
                                                 0, 0)),
            pl.BlockSpec((1, 1, D),
                         lambda t, off, mi, gi: (jnp.minimum(gi[t], E - 1),
                                                 0, 0)),
        ],
        out_specs=pl.BlockSpec((bm, D), lambda t, off, mi, gi: (mi[t], 0)),
    )
    return pl.pallas_call(
        functools.partial(_gmm_body, bm),
        grid_spec=grid_spec,
        out_shape=jax.ShapeDtypeStruct((ntot, D), jnp.float32),
        compiler_params=pltpu.CompilerParams(
            vmem_limit_bytes=100 * 1024 * 1024),
    )(offsets, m_ids, g_ids, xs, w1, b1.reshape(E, 1, F), w2,
      b2.reshape(E, 1, D))


def _combine_body(x_ref, y_ref, w_ref, o_ref):
    w0 = w_ref[:, 0:1]
    w1 = w_ref[:, 1:2]
    o_ref[...] = x_ref[...] + w0 * y_ref[:, 0, :] + w1 * y_ref[:, 1, :]


def _combine(x, y2, w):
    n = x.shape[0]
    bn = min(n, 256)
    return pl.pallas_call(
        _combine_body,
        grid=(n // bn,),
        in_specs=[
            pl.BlockSpec((bn, D), lambda i: (i, 0)),
            pl.BlockSpec((bn, 2, D), lambda i: (i, 0, 0)),
            pl.BlockSpec((bn, 2), lambda i: (i, 0)),
        ],
        out_specs=pl.BlockSpec((bn, D), lambda i: (i, 0)),
        out_shape=jax.ShapeDtypeStruct((n, D), jnp.float32),
    )(x, y2, w)


def _add_body(a_ref, b_ref, o_ref):
    o_ref[...] = a_ref[...] + b_ref[...]


def _addk(a, b):
    n = a.shape[0]
    bn = min(n, 256)
    return pl.pallas_call(
        _add_body,
        grid=(n // bn,),
        in_specs=[pl.BlockSpec((bn, D), lambda i: (i, 0))] * 2,
        out_specs=pl.BlockSpec((bn, D), lambda i: (i, 0)),
        out_shape=jax.ShapeDtypeStruct((n, D), jnp.float32),
    )(a, b)


def _layer(x, p, n):
    h1 = _ln(x, p['ln1_g'], p['ln1_b'])
    qkv = _matmul(h1, p['attn_in_w'], p['attn_in_b'], bm=768)
    q = qkv[:, :D].reshape(n, H, DH).transpose(1, 0, 2)
    k = qkv[:, D:2 * D].reshape(n, H, DH).transpose(1, 0, 2)
    v = qkv[:, 2 * D:].reshape(n, H, DH).transpose(1, 0, 2)
    o = _attn(q, k, v).transpose(1, 0, 2).reshape(n, D)
    x = _matmul(o, p['attn_out_w'], p['attn_out_b'], bm=768, residual=x)
    h2 = _ln(x, p['ln2_g'], p['ln2_b'])
    rw = jnp.zeros((128, D), jnp.float32).at[:E].set(p['router_w'])
    logits = _matmul(h2, rw, jnp.zeros((128,), jnp.float32), bm=128)
    idx2, w2, stats = _router(logits, n)
    ntot = 2 * n
    st, pos, off = _dispatch(idx2.reshape(ntot), ntot)
    xs = _gather(h2, st)
    ys = _gmm(xs, p['e_w1'], p['e_b1'], p['e_w2'], p['e_b2'], off, ntot)
    y2 = _gather(ys, pos).reshape(n, 2, D)
    x = _combine(x, y2, w2)
    return x, stats[2, 0]


def kernel(x_emb, params):
    b, t, _ = x_emb.shape
    x = _addk(x_emb[0], params['pos'][:t])
    bal = jnp.float32(0.0)
    h = x
    for p in params['local']:
        h, bl = _layer(h, p, t)
        bal = bal + bl
    syn = h[15::16]
    s = syn.shape[0]
    g = syn
    for p in params['global']:
        g, bl = _layer(g, p, s)
        bal = bal + bl
    rep = jnp.repeat(g, 16, axis=0)
    out = _ln(h, params['ln_g'], params['ln_b'], residual=rep)
    logits = _matmul(out, params['head_w'], jnp.zeros((8192,), jnp.float32),
                     bm=512)
    return logits[None], bal
